# baseline (speedup 1.0000x reference)
import jax
import jax.numpy as jnp
from jax.experimental import pallas as pl

_N, _D = 1_000_000, 64
_BC = 40_960


def _body(x_ref, o_ref):
    o_ref[...] = x_ref[...] + 1.0

    @pl.when(pl.program_id(0) == 0)
    def _fix_col1():
        o_ref[:, 1] = jnp.full((_D,), 2.0, dtype=o_ref.dtype)


def kernel(x):
    xt = x.T
    grid = (_N + _BC - 1) // _BC
    out_t = pl.pallas_call(
        _body,
        grid=(grid,),
        in_specs=[pl.BlockSpec((_D, _BC), lambda j: (0, j))],
        out_specs=pl.BlockSpec((_D, _BC), lambda j: (0, j)),
        out_shape=jax.ShapeDtypeStruct((_D, _N), jnp.float32),
    )(xt)
    return out_t.T


# speedup vs baseline: 1.0005x; 1.0005x over previous
import jax
import jax.numpy as jnp
from jax.experimental import pallas as pl

_BC = 40_960


def _body(x_ref, o_ref):
    o_ref[...] = x_ref[...] + 1.0

    @pl.when(pl.program_id(0) == 0)
    def _fix_col1():
        o_ref[:, 1] = jnp.full((o_ref.shape[0],), 2.0, dtype=o_ref.dtype)


def kernel(x):
    n, d = x.shape
    xt = x.T
    grid = (n + _BC - 1) // _BC
    out_t = pl.pallas_call(
        _body,
        grid=(grid,),
        in_specs=[pl.BlockSpec((d, _BC), lambda j: (0, j))],
        out_specs=pl.BlockSpec((d, _BC), lambda j: (0, j)),
        out_shape=jax.ShapeDtypeStruct((d, n), x.dtype),
    )(xt)
    return out_t.T
